# Initial kernel scaffold; baseline (speedup 1.0000x reference)
import jax
import jax.numpy as jnp
from jax.experimental import pallas as pl


def kernel(x, edge_index, Wl1, bl1, Wr1, Wl2, bl2, Wr2):
    raise NotImplementedError("write your pallas kernel here")


# speedup vs baseline: 3.4931x; 3.4931x over previous
import functools

import jax
import jax.numpy as jnp
from jax import lax
from jax.experimental import pallas as pl
from jax.experimental.pallas import tpu as pltpu
from jax.experimental.pallas import tpu_sc as plsc

N = 10000
D = 128
E = 320000

NC, NS = 2, 16
NW = NC * NS
CHUNK = 64
K = 160
IG = 32
NG = K // IG
EPW = K * CHUNK
E_PAD = NW * EPW
N_PAD = 10240
RPT = N_PAD // NS
D1 = D + 16
PAD_DST = N
NBUF = 2


def _make_sc_aggregate(width):
    mesh = plsc.VectorSubcoreMesh(
        core_axis_name="c", subcore_axis_name="s", num_cores=NC, num_subcores=NS
    )

    @functools.partial(
        pl.kernel,
        out_type=jax.ShapeDtypeStruct((NC * N_PAD, width), jnp.float32),
        mesh=mesh,
        compiler_params=pltpu.CompilerParams(use_tc_tiling_on_sc=False),
        scratch_types=[
            pltpu.VMEM((IG, CHUNK), jnp.int32),
            pltpu.VMEM((IG, CHUNK), jnp.int32),
            pltpu.VMEM((NBUF, CHUNK, width), jnp.float32),
            pltpu.VMEM_SHARED((N_PAD, width), jnp.float32),
            pltpu.SemaphoreType.DMA,
            pltpu.SemaphoreType.DMA,
        ],
    )
    def agg(table, srcidx, dstidx, zrows, out, sidx_v, didx_v, rows_v, acc_sh,
            sem0, sem1):
        c = lax.axis_index("c")
        s = lax.axis_index("s")
        wid = s * NC + c
        sems = [sem0, sem1]

        pltpu.sync_copy(zrows, acc_sh.at[pl.ds(s * RPT, RPT)])
        plsc.subcore_barrier()

        @pl.loop(0, NG)
        def _(g):
            base = wid * K + g * IG
            pltpu.sync_copy(srcidx.at[pl.ds(base, IG)], sidx_v)
            pltpu.sync_copy(dstidx.at[pl.ds(base, IG)], didx_v)

            for b in range(NBUF):
                pltpu.async_copy(table.at[sidx_v.at[b]], rows_v.at[b], sems[b])

            @pl.loop(0, IG, step=NBUF)
            def _(t):
                for b in range(NBUF):
                    j = t + b
                    pltpu.make_async_copy(
                        table.at[sidx_v.at[j]], rows_v.at[b], sems[b]
                    ).wait()
                    pltpu.sync_copy(
                        rows_v.at[b], acc_sh.at[didx_v.at[j]], add=True
                    )

                    @pl.when(j + NBUF < IG)
                    def _():
                        pltpu.async_copy(
                            table.at[sidx_v.at[j + NBUF]], rows_v.at[b], sems[b]
                        )

        plsc.subcore_barrier()
        pltpu.sync_copy(
            acc_sh.at[pl.ds(s * RPT, RPT)],
            out.at[pl.ds(c * N_PAD + s * RPT, RPT)],
        )

    return agg


_sc_aggregate_d1 = _make_sc_aggregate(D1)
_sc_aggregate_d = _make_sc_aggregate(D)

M_BLK = 1024


def _tc_layer1_body(p_ref, x_ref, wl_ref, bl_ref, wr_ref, h_ref, r_ref):
    ps = p_ref[0] + p_ref[1]
    agg = ps[:, :D]
    deg = ps[:, D:D + 1]
    r = 1.0 / jnp.maximum(deg, 1.0)
    h = (
        jnp.dot(agg * r, wl_ref[...], preferred_element_type=jnp.float32)
        + bl_ref[...]
        + jnp.dot(x_ref[...][:, :D], wr_ref[...], preferred_element_type=jnp.float32)
    )
    h_ref[...] = jnp.maximum(h, 0.0)
    r_ref[...] = jnp.broadcast_to(r, (M_BLK, D))


def _tc_layer2_body(p_ref, h_ref, r_ref, wl_ref, bl_ref, wr_ref, o_ref):
    agg = p_ref[0] + p_ref[1]
    r = r_ref[...]
    o_ref[...] = (
        jnp.dot(agg * r, wl_ref[...], preferred_element_type=jnp.float32)
        + bl_ref[...]
        + jnp.dot(h_ref[...], wr_ref[...], preferred_element_type=jnp.float32)
    )


def _tc_layer1(p, x_aug, wlT, bl, wrT):
    grid = (N_PAD // M_BLK,)
    return pl.pallas_call(
        _tc_layer1_body,
        grid=grid,
        in_specs=[
            pl.BlockSpec((NC, M_BLK, D1), lambda i: (0, i, 0)),
            pl.BlockSpec((M_BLK, D1), lambda i: (i, 0)),
            pl.BlockSpec((D, D), lambda i: (0, 0)),
            pl.BlockSpec((1, D), lambda i: (0, 0)),
            pl.BlockSpec((D, D), lambda i: (0, 0)),
        ],
        out_specs=[
            pl.BlockSpec((M_BLK, D), lambda i: (i, 0)),
            pl.BlockSpec((M_BLK, 128), lambda i: (i, 0)),
        ],
        out_shape=[
            jax.ShapeDtypeStruct((N_PAD, D), jnp.float32),
            jax.ShapeDtypeStruct((N_PAD, 128), jnp.float32),
        ],
    )(p, x_aug, wlT, bl, wrT)


def _tc_layer2(p, h1, r2d, wlT, bl, wrT):
    grid = (N_PAD // M_BLK,)
    return pl.pallas_call(
        _tc_layer2_body,
        grid=grid,
        in_specs=[
            pl.BlockSpec((NC, M_BLK, D), lambda i: (0, i, 0)),
            pl.BlockSpec((M_BLK, D), lambda i: (i, 0)),
            pl.BlockSpec((M_BLK, 128), lambda i: (i, 0)),
            pl.BlockSpec((D, D), lambda i: (0, 0)),
            pl.BlockSpec((1, D), lambda i: (0, 0)),
            pl.BlockSpec((D, D), lambda i: (0, 0)),
        ],
        out_specs=pl.BlockSpec((M_BLK, D), lambda i: (i, 0)),
        out_shape=jax.ShapeDtypeStruct((N_PAD, D), jnp.float32),
    )(p, h1, r2d, wlT, bl, wrT)


def kernel(x, edge_index, Wl1, bl1, Wr1, Wl2, bl2, Wr2):
    src = edge_index[0].astype(jnp.int32)
    dst = edge_index[1].astype(jnp.int32)
    src_p = jnp.concatenate([src, jnp.zeros((E_PAD - E,), jnp.int32)])
    dst_p = jnp.concatenate([dst, jnp.full((E_PAD - E,), PAD_DST, jnp.int32)])
    src_r = src_p.reshape(NW * K, CHUNK)
    dst_r = dst_p.reshape(NW * K, CHUNK)

    x_aug = jnp.zeros((N_PAD, D1), jnp.float32)
    x_aug = x_aug.at[:N, :D].set(x)
    x_aug = x_aug.at[:N, D:].set(1.0)

    zrows1 = jnp.zeros((RPT, D1), jnp.float32)
    zrows2 = jnp.zeros((RPT, D), jnp.float32)

    p1 = _sc_aggregate_d1(x_aug, src_r, dst_r, zrows1).reshape(NC, N_PAD, D1)
    h1, r2d = _tc_layer1(p1, x_aug, Wl1.T, bl1.reshape(1, D), Wr1.T)

    p2 = _sc_aggregate_d(h1, src_r, dst_r, zrows2).reshape(NC, N_PAD, D)
    out = _tc_layer2(p2, h1, r2d, Wl2.T, bl2.reshape(1, D), Wr2.T)

    return out[:N]
